# Initial kernel scaffold; baseline (speedup 1.0000x reference)
import jax
import jax.numpy as jnp
from jax.experimental import pallas as pl


def kernel(betas, alphas, alpha_bars, num_steps):
    raise NotImplementedError("write your pallas kernel here")


# speedup vs baseline: 11.8127x; 11.8127x over previous
import jax
import jax.numpy as jnp
from jax import lax
from jax.experimental import pallas as pl
from jax.experimental.pallas import tpu as pltpu
from jax.experimental.pallas import tpu_sc as plsc

MAX_STEPS = 1000
BATCH = 16384
NC = 2
NS = 16
NW = NC * NS
B_PER_W = BATCH // NW
LANES = 16


def _body(betas_hbm, alphas_hbm, abars_hbm, idx_hbm,
          outb_hbm, outa_hbm, outab_hbm,
          tab_b, tab_a, tab_ab, idx_v, ob_v, oa_v, oab_v):
    wid = lax.axis_index("s") * NC + lax.axis_index("c")
    base = wid * B_PER_W
    pltpu.sync_copy(betas_hbm, tab_b)
    pltpu.sync_copy(alphas_hbm, tab_a)
    pltpu.sync_copy(abars_hbm, tab_ab)
    pltpu.sync_copy(idx_hbm.at[pl.ds(base, B_PER_W)], idx_v)
    for i in range(B_PER_W // LANES):
        sl = pl.ds(i * LANES, LANES)
        idx = idx_v[sl]
        ob_v[sl] = plsc.load_gather(tab_b, [idx])
        oa_v[sl] = plsc.load_gather(tab_a, [idx])
        oab_v[sl] = plsc.load_gather(tab_ab, [idx])
    pltpu.sync_copy(ob_v, outb_hbm.at[pl.ds(base, B_PER_W)])
    pltpu.sync_copy(oa_v, outa_hbm.at[pl.ds(base, B_PER_W)])
    pltpu.sync_copy(oab_v, outab_hbm.at[pl.ds(base, B_PER_W)])


def kernel(betas, alphas, alpha_bars, num_steps):
    f32 = jnp.float32
    out = jax.ShapeDtypeStruct((BATCH,), f32)
    k = pl.kernel(
        _body,
        out_type=(out, out, out),
        mesh=plsc.VectorSubcoreMesh(core_axis_name="c", subcore_axis_name="s"),
        compiler_params=pltpu.CompilerParams(needs_layout_passes=False),
        scratch_types=[
            pltpu.VMEM((MAX_STEPS,), f32),
            pltpu.VMEM((MAX_STEPS,), f32),
            pltpu.VMEM((MAX_STEPS,), f32),
            pltpu.VMEM((B_PER_W,), jnp.int32),
            pltpu.VMEM((B_PER_W,), f32),
            pltpu.VMEM((B_PER_W,), f32),
            pltpu.VMEM((B_PER_W,), f32),
        ],
    )
    return k(betas, alphas, alpha_bars, num_steps.astype(jnp.int32))


# speedup vs baseline: 12.4040x; 1.0501x over previous
import jax
import jax.numpy as jnp
from jax import lax
from jax.experimental import pallas as pl
from jax.experimental.pallas import tpu as pltpu
from jax.experimental.pallas import tpu_sc as plsc

MAX_STEPS = 1000
BATCH = 16384
NC = 2
NS = 16
NW = NC * NS
B_PER_W = BATCH // NW
LANES = 16


def _body(betas_hbm, alphas_hbm, abars_hbm, idx_hbm,
          outb_hbm, outa_hbm, outab_hbm,
          tab_b, tab_a, tab_ab, idx_v, ob_v, oa_v, oab_v, sem):
    wid = lax.axis_index("s") * NC + lax.axis_index("c")
    base = wid * B_PER_W
    c0 = pltpu.async_copy(betas_hbm, tab_b, sem)
    c1 = pltpu.async_copy(alphas_hbm, tab_a, sem)
    c2 = pltpu.async_copy(abars_hbm, tab_ab, sem)
    c3 = pltpu.async_copy(idx_hbm.at[pl.ds(base, B_PER_W)], idx_v, sem)
    c0.wait(); c1.wait(); c2.wait(); c3.wait()
    for i in range(B_PER_W // LANES):
        sl = pl.ds(i * LANES, LANES)
        idx = idx_v[sl]
        ob_v[sl] = plsc.load_gather(tab_b, [idx])
        oa_v[sl] = plsc.load_gather(tab_a, [idx])
        oab_v[sl] = plsc.load_gather(tab_ab, [idx])
    o0 = pltpu.async_copy(ob_v, outb_hbm.at[pl.ds(base, B_PER_W)], sem)
    o1 = pltpu.async_copy(oa_v, outa_hbm.at[pl.ds(base, B_PER_W)], sem)
    o2 = pltpu.async_copy(oab_v, outab_hbm.at[pl.ds(base, B_PER_W)], sem)
    o0.wait(); o1.wait(); o2.wait()


def kernel(betas, alphas, alpha_bars, num_steps):
    f32 = jnp.float32
    out = jax.ShapeDtypeStruct((BATCH,), f32)
    k = pl.kernel(
        _body,
        out_type=(out, out, out),
        mesh=plsc.VectorSubcoreMesh(core_axis_name="c", subcore_axis_name="s"),
        compiler_params=pltpu.CompilerParams(needs_layout_passes=False),
        scratch_types=[
            pltpu.VMEM((MAX_STEPS,), f32),
            pltpu.VMEM((MAX_STEPS,), f32),
            pltpu.VMEM((MAX_STEPS,), f32),
            pltpu.VMEM((B_PER_W,), jnp.int32),
            pltpu.VMEM((B_PER_W,), f32),
            pltpu.VMEM((B_PER_W,), f32),
            pltpu.VMEM((B_PER_W,), f32),
            pltpu.SemaphoreType.DMA,
        ],
    )
    return k(betas, alphas, alpha_bars, num_steps.astype(jnp.int32))
